# baseline (speedup 1.0000x reference)
import functools
import jax
import jax.numpy as jnp
from jax import lax
from jax.experimental import pallas as pl
from jax.experimental.pallas import tpu as pltpu, tpu_sc as plsc

N = 10752
E = 903168
G = 128
NPAD = 10880
NBLK = 8
BLKROWS = NPAD // NBLK
W = 128

NC, NS = 2, 16
NW = NC * NS
ET = E + N
EPT = ET // NW
C1 = 112
C2 = 80
NCH1 = EPT // C1
NCH2 = EPT // C2

ROWS_PT = NPAD // NS
AW = 128


def _zero_acc(outb, acc_sh, s):
    rows = outb.shape[0]

    @pl.loop(0, rows)
    def _rows(i):
        for j in range(AW // 16):
            outb[i, pl.ds(16 * j, 16)] = jnp.zeros((16,), jnp.float32)
    done = 0
    while done < ROWS_PT:
        r = min(rows, ROWS_PT - done)
        pltpu.sync_copy(outb.at[pl.ds(0, r)],
                        acc_sh.at[pl.ds(s * ROWS_PT + done, r)])
        done += r
    plsc.subcore_barrier()


def _writeback(acc_sh, out_hbm, c, s):
    plsc.subcore_barrier()
    pltpu.sync_copy(acc_sh.at[pl.ds(s * ROWS_PT, ROWS_PT)],
                    out_hbm.at[c, pl.ds(s * ROWS_PT, ROWS_PT)])


IDXB = 3


def _edge_pass(t_hbm, src_hbm, dst_hbm, out_hbm,
               idxs_big, idxd_big, idx_d, inb, dnb, outb, acc_sh, sem0, sem1,
               *, body):
    c = lax.axis_index("c")
    s = lax.axis_index("s")
    _zero_acc(outb, acc_sh, s)
    base_e = (c * NS + s) * EPT

    @pl.loop(0, NCH1 // IDXB)
    def _outer(gg):
        gbase = base_e + gg * (IDXB * C1)
        pltpu.sync_copy(src_hbm.at[pl.ds(gbase, IDXB * C1)], idxs_big)
        pltpu.sync_copy(dst_hbm.at[pl.ds(gbase, IDXB * C1)], idxd_big)
        for u in range(IDXB):
            for v in range(C1 // 16):
                idx_d[pl.ds(v * 16, 16)] = idxd_big[pl.ds(u * C1 + v * 16, 16)]
            cp0 = pltpu.async_copy(
                t_hbm.at[idxs_big.at[pl.ds(u * C1, C1)]], inb, sem0)
            cp1 = pltpu.async_copy(
                t_hbm.at[idxd_big.at[pl.ds(u * C1, C1)]], dnb, sem1)
            cp0.wait()
            cp1.wait()

            @pl.loop(0, C1, unroll=2)
            def _edges(e):
                body(e, inb, dnb, outb)

            pltpu.sync_copy(outb, acc_sh.at[idx_d], add=True)

    _writeback(acc_sh, out_hbm, c, s)


def _edge_body1(e, inb, dnb, outb):
    as0 = inb[e, pl.ds(64, 16)]
    as1 = inb[e, pl.ds(80, 16)]
    ad0 = dnb[e, pl.ds(96, 16)]
    ad1 = dnb[e, pl.ds(112, 16)]
    t0 = as0 + ad0
    t1 = as1 + ad1
    s0 = jnp.exp(jnp.maximum(t0, t0 * 0.2))
    s1 = jnp.exp(jnp.maximum(t1, t1 * 0.2))
    outb[e, pl.ds(0, 16)] = inb[e, pl.ds(0, 16)] * s0
    outb[e, pl.ds(16, 16)] = inb[e, pl.ds(16, 16)] * s1
    outb[e, pl.ds(32, 16)] = inb[e, pl.ds(32, 16)] * s0
    outb[e, pl.ds(48, 16)] = inb[e, pl.ds(48, 16)] * s1
    outb[e, pl.ds(64, 16)] = s0
    outb[e, pl.ds(80, 16)] = s1


def _edge_body2(e, inb, dnb, outb):
    t = inb[e, pl.ds(64, 16)] + dnb[e, pl.ds(80, 16)]
    s = jnp.exp(jnp.maximum(t, t * 0.2))
    for k in range(4):
        outb[e, pl.ds(16 * k, 16)] = inb[e, pl.ds(16 * k, 16)] * s
    outb[e, pl.ds(64, 16)] = s


def _make_edge_kernel(body):
    mesh = plsc.VectorSubcoreMesh(core_axis_name="c", subcore_axis_name="s",
                                  num_cores=NC, num_subcores=NS)
    return pl.kernel(
        functools.partial(_edge_pass, body=body),
        out_type=jax.ShapeDtypeStruct((NC, NPAD, AW), jnp.float32),
        mesh=mesh,
        scratch_types=[
            pltpu.VMEM((IDXB * C1,), jnp.int32),
            pltpu.VMEM((IDXB * C1,), jnp.int32),
            pltpu.VMEM((C1,), jnp.int32),
            pltpu.VMEM((C1, W), jnp.float32),
            pltpu.VMEM((C1, W), jnp.float32),
            pltpu.VMEM((C1, AW), jnp.float32),
            pltpu.VMEM_SHARED((NPAD, AW), jnp.float32),
            pltpu.SemaphoreType.DMA,
            pltpu.SemaphoreType.DMA,
        ],
    )


def _prep1_body(x_ref, w_ref, asrc_ref, adst_ref, t1_ref):
    h = jnp.dot(x_ref[...], w_ref[...], preferred_element_type=jnp.float32)
    ha = h * asrc_ref[...]
    as_ = ha[:, :32] + ha[:, 32:]
    hd = h * adst_ref[...]
    ad_ = hd[:, :32] + hd[:, 32:]
    t1_ref[...] = jnp.concatenate([h, as_, ad_], axis=1)


def _comb1_body(acc_ref, w2_ref, b1_ref, asrc_ref, adst_ref, t2_ref):
    num = acc_ref[0, :, :64] + acc_ref[1, :, :64]
    den = acc_ref[0, :, 64:96] + acc_ref[1, :, 64:96]
    den64 = jnp.concatenate([den, den], axis=1)
    h1 = num / (den64 + 1e-16) + b1_ref[...]
    h1 = jnp.where(h1 > 0, h1, jnp.exp(jnp.minimum(h1, 0.0)) - 1.0)
    h2 = jnp.dot(h1, w2_ref[...], preferred_element_type=jnp.float32)
    a_s = jnp.sum(h2 * asrc_ref[...], axis=1, keepdims=True)
    a_d = jnp.sum(h2 * adst_ref[...], axis=1, keepdims=True)
    n = h2.shape[0]
    t2_ref[...] = jnp.concatenate(
        [h2, jnp.broadcast_to(a_s, (n, 16)), jnp.broadcast_to(a_d, (n, 16)),
         jnp.zeros((n, 32), jnp.float32)], axis=1)


def _pool_body(acc_ref, batch_ref, b2_ref, fcw_ref, fcb_ref, out_ref, s_ref):
    i = pl.program_id(0)

    @pl.when(i == 0)
    def _():
        s_ref[...] = jnp.zeros_like(s_ref)

    num = acc_ref[0, :, :64] + acc_ref[1, :, :64]
    den = acc_ref[0, :, 64:65] + acc_ref[1, :, 64:65]
    h2 = num / (den + 1e-16) + b2_ref[...]
    h2 = jnp.where(h2 > 0, h2, jnp.exp(jnp.minimum(h2, 0.0)) - 1.0)
    hw = jnp.concatenate([h2, jnp.ones((h2.shape[0], 1), jnp.float32)], axis=1)
    b = batch_ref[0]
    iota = lax.broadcasted_iota(jnp.int32, (G, h2.shape[0]), 0)
    oh = (b == iota).astype(jnp.float32)
    s_ref[...] += jnp.dot(oh, hw, preferred_element_type=jnp.float32)

    @pl.when(i == pl.num_programs(0) - 1)
    def _():
        pooled = s_ref[:, :64] / jnp.maximum(s_ref[:, 64:65], 1.0)
        out_ref[...] = jnp.dot(pooled, fcw_ref[...],
                               preferred_element_type=jnp.float32) + fcb_ref[...]


def kernel(x, edge_index, batch, W1, att_src1, att_dst1, b1,
           W2, att_src2, att_dst2, b2, fcW, fcb):
    f32 = jnp.float32
    W1p = W1.reshape(84, 32, 2).transpose(0, 2, 1).reshape(84, 64)
    W1pp = jnp.pad(W1p, ((0, 44), (0, 0)))
    A1s = att_src1.T.reshape(1, 64)
    A1d = att_dst1.T.reshape(1, 64)
    b1p = b1.reshape(32, 2).T.reshape(1, 64)
    W2p = W2.reshape(32, 2, 64).transpose(1, 0, 2).reshape(64, 64)
    A2s = att_src2.reshape(1, 64)
    A2d = att_dst2.reshape(1, 64)
    b2r = b2.reshape(1, 64)

    x_pad = jnp.pad(x, ((0, NPAD - N), (0, 128 - x.shape[1])))

    loop = jnp.arange(N, dtype=jnp.int32)
    src = jnp.concatenate([edge_index[0].astype(jnp.int32), loop])
    dst = jnp.concatenate([edge_index[1].astype(jnp.int32), loop])
    batch_pad = jnp.concatenate(
        [batch.astype(jnp.int32), jnp.full((NPAD - N,), G, jnp.int32)]
    ).reshape(NBLK, 1, BLKROWS)

    t1 = pl.pallas_call(
        _prep1_body,
        grid=(NBLK,),
        in_specs=[
            pl.BlockSpec((BLKROWS, 128), lambda i: (i, 0)),
            pl.BlockSpec((128, 64), lambda i: (0, 0)),
            pl.BlockSpec((1, 64), lambda i: (0, 0)),
            pl.BlockSpec((1, 64), lambda i: (0, 0)),
        ],
        out_specs=pl.BlockSpec((BLKROWS, W), lambda i: (i, 0)),
        out_shape=jax.ShapeDtypeStruct((NPAD, W), f32),
    )(x_pad, W1pp, A1s, A1d)

    acc1 = _make_edge_kernel(_edge_body1)(t1, src, dst)

    t2 = pl.pallas_call(
        _comb1_body,
        grid=(NBLK,),
        in_specs=[
            pl.BlockSpec((2, BLKROWS, AW), lambda i: (0, i, 0)),
            pl.BlockSpec((64, 64), lambda i: (0, 0)),
            pl.BlockSpec((1, 64), lambda i: (0, 0)),
            pl.BlockSpec((1, 64), lambda i: (0, 0)),
            pl.BlockSpec((1, 64), lambda i: (0, 0)),
        ],
        out_specs=pl.BlockSpec((BLKROWS, W), lambda i: (i, 0)),
        out_shape=jax.ShapeDtypeStruct((NPAD, W), f32),
    )(acc1, W2p, b1p, A2s, A2d)

    acc2 = _make_edge_kernel(_edge_body2)(t2, src, dst)

    out = pl.pallas_call(
        _pool_body,
        grid=(NBLK,),
        in_specs=[
            pl.BlockSpec((2, BLKROWS, AW), lambda i: (0, i, 0)),
            pl.BlockSpec((1, 1, BLKROWS), lambda i: (i, 0, 0)),
            pl.BlockSpec((1, 64), lambda i: (0, 0)),
            pl.BlockSpec((64, 1), lambda i: (0, 0)),
            pl.BlockSpec((1, 1), lambda i: (0, 0)),
        ],
        out_specs=pl.BlockSpec((G, 1), lambda i: (0, 0)),
        out_shape=jax.ShapeDtypeStruct((G, 1), f32),
        scratch_shapes=[pltpu.VMEM((G, 65), f32)],
    )(acc2, batch_pad, b2r, fcW, fcb.reshape(1, 1))

    z = jnp.zeros((1,), f32)
    return (out, z, z, z, z, z, z)


# speedup vs baseline: 1.7737x; 1.7737x over previous
import functools
import jax
import jax.numpy as jnp
from jax import lax
from jax.experimental import pallas as pl
from jax.experimental.pallas import tpu as pltpu, tpu_sc as plsc

N = 10752
E = 903168
G = 128
NPAD = 10880
NBLK = 8
BLKROWS = NPAD // NBLK
W = 128

NC, NS = 2, 16
NW = NC * NS
ET = E + N
EPT = ET // NW
C1 = 112
C2 = 80
NCH1 = EPT // C1
NCH2 = EPT // C2

ROWS_PT = NPAD // NS
AW = 128


def _zero_acc(outb, acc_sh, s):
    rows = outb.shape[0]

    @pl.loop(0, rows)
    def _rows(i):
        for j in range(AW // 16):
            outb[i, pl.ds(16 * j, 16)] = jnp.zeros((16,), jnp.float32)
    done = 0
    while done < ROWS_PT:
        r = min(rows, ROWS_PT - done)
        pltpu.sync_copy(outb.at[pl.ds(0, r)],
                        acc_sh.at[pl.ds(s * ROWS_PT + done, r)])
        done += r
    plsc.subcore_barrier()


def _writeback(acc_sh, out_hbm, c, s):
    plsc.subcore_barrier()
    pltpu.sync_copy(acc_sh.at[pl.ds(s * ROWS_PT, ROWS_PT)],
                    out_hbm.at[c, pl.ds(s * ROWS_PT, ROWS_PT)])


IDXB = 3


def _edge_pass(t_hbm, src_hbm, dst_hbm, out_hbm,
               idxs_big, idxd_big, idx_d, inb, dnb, outb, acc_sh, sem0, sem1,
               *, body):
    c = lax.axis_index("c")
    s = lax.axis_index("s")
    _zero_acc(outb, acc_sh, s)
    base_e = (c * NS + s) * EPT

    @pl.loop(0, NCH1 // IDXB)
    def _outer(gg):
        gbase = base_e + gg * (IDXB * C1)
        pltpu.sync_copy(src_hbm.at[pl.ds(gbase, IDXB * C1)], idxs_big)
        pltpu.sync_copy(dst_hbm.at[pl.ds(gbase, IDXB * C1)], idxd_big)
        for u in range(IDXB):
            for v in range(C1 // 16):
                idx_d[pl.ds(v * 16, 16)] = idxd_big[pl.ds(u * C1 + v * 16, 16)]
            cp0 = pltpu.async_copy(
                t_hbm.at[idxs_big.at[pl.ds(u * C1, C1)]], inb, sem0)
            cp1 = pltpu.async_copy(
                t_hbm.at[idxd_big.at[pl.ds(u * C1, C1)]], dnb, sem1)
            cp0.wait()
            cp1.wait()

            @pl.loop(0, C1)
            def _edges(e):
                body(e, inb, dnb, outb)

            pltpu.sync_copy(outb, acc_sh.at[idx_d], add=True)

    _writeback(acc_sh, out_hbm, c, s)


def _edge_body1(e, inb, dnb, outb):
    as0 = inb[e, pl.ds(64, 16)]
    as1 = inb[e, pl.ds(80, 16)]
    ad0 = dnb[e, pl.ds(96, 16)]
    ad1 = dnb[e, pl.ds(112, 16)]
    t0 = as0 + ad0
    t1 = as1 + ad1
    s0 = jnp.exp(jnp.maximum(t0, t0 * 0.2))
    s1 = jnp.exp(jnp.maximum(t1, t1 * 0.2))
    outb[e, pl.ds(0, 16)] = inb[e, pl.ds(0, 16)] * s0
    outb[e, pl.ds(16, 16)] = inb[e, pl.ds(16, 16)] * s1
    outb[e, pl.ds(32, 16)] = inb[e, pl.ds(32, 16)] * s0
    outb[e, pl.ds(48, 16)] = inb[e, pl.ds(48, 16)] * s1
    outb[e, pl.ds(64, 16)] = s0
    outb[e, pl.ds(80, 16)] = s1


def _edge_body2(e, inb, dnb, outb):
    t = inb[e, pl.ds(64, 16)] + dnb[e, pl.ds(80, 16)]
    s = jnp.exp(jnp.maximum(t, t * 0.2))
    for k in range(4):
        outb[e, pl.ds(16 * k, 16)] = inb[e, pl.ds(16 * k, 16)] * s
    outb[e, pl.ds(64, 16)] = s


def _make_edge_kernel(body):
    mesh = plsc.VectorSubcoreMesh(core_axis_name="c", subcore_axis_name="s",
                                  num_cores=NC, num_subcores=NS)
    return pl.kernel(
        functools.partial(_edge_pass, body=body),
        out_type=jax.ShapeDtypeStruct((NC, NPAD, AW), jnp.float32),
        mesh=mesh,
        scratch_types=[
            pltpu.VMEM((IDXB * C1,), jnp.int32),
            pltpu.VMEM((IDXB * C1,), jnp.int32),
            pltpu.VMEM((C1,), jnp.int32),
            pltpu.VMEM((C1, W), jnp.float32),
            pltpu.VMEM((C1, W), jnp.float32),
            pltpu.VMEM((C1, AW), jnp.float32),
            pltpu.VMEM_SHARED((NPAD, AW), jnp.float32),
            pltpu.SemaphoreType.DMA,
            pltpu.SemaphoreType.DMA,
        ],
    )


def _prep1_body(x_ref, w_ref, asrc_ref, adst_ref, t1_ref):
    h = jnp.dot(x_ref[...], w_ref[...], preferred_element_type=jnp.float32)
    ha = h * asrc_ref[...]
    as_ = ha[:, :32] + ha[:, 32:]
    hd = h * adst_ref[...]
    ad_ = hd[:, :32] + hd[:, 32:]
    t1_ref[...] = jnp.concatenate([h, as_, ad_], axis=1)


def _comb1_body(acc_ref, w2_ref, b1_ref, asrc_ref, adst_ref, t2_ref):
    num = acc_ref[0, :, :64] + acc_ref[1, :, :64]
    den = acc_ref[0, :, 64:96] + acc_ref[1, :, 64:96]
    den64 = jnp.concatenate([den, den], axis=1)
    h1 = num / (den64 + 1e-16) + b1_ref[...]
    h1 = jnp.where(h1 > 0, h1, jnp.exp(jnp.minimum(h1, 0.0)) - 1.0)
    h2 = jnp.dot(h1, w2_ref[...], preferred_element_type=jnp.float32)
    a_s = jnp.sum(h2 * asrc_ref[...], axis=1, keepdims=True)
    a_d = jnp.sum(h2 * adst_ref[...], axis=1, keepdims=True)
    n = h2.shape[0]
    t2_ref[...] = jnp.concatenate(
        [h2, jnp.broadcast_to(a_s, (n, 16)), jnp.broadcast_to(a_d, (n, 16)),
         jnp.zeros((n, 32), jnp.float32)], axis=1)


def _pool_body(acc_ref, batch_ref, b2_ref, fcw_ref, fcb_ref, out_ref, s_ref):
    i = pl.program_id(0)

    @pl.when(i == 0)
    def _():
        s_ref[...] = jnp.zeros_like(s_ref)

    num = acc_ref[0, :, :64] + acc_ref[1, :, :64]
    den = acc_ref[0, :, 64:65] + acc_ref[1, :, 64:65]
    h2 = num / (den + 1e-16) + b2_ref[...]
    h2 = jnp.where(h2 > 0, h2, jnp.exp(jnp.minimum(h2, 0.0)) - 1.0)
    hw = jnp.concatenate([h2, jnp.ones((h2.shape[0], 1), jnp.float32)], axis=1)
    b = batch_ref[0]
    iota = lax.broadcasted_iota(jnp.int32, (G, h2.shape[0]), 0)
    oh = (b == iota).astype(jnp.float32)
    s_ref[...] += jnp.dot(oh, hw, preferred_element_type=jnp.float32)

    @pl.when(i == pl.num_programs(0) - 1)
    def _():
        pooled = s_ref[:, :64] / jnp.maximum(s_ref[:, 64:65], 1.0)
        out_ref[...] = jnp.dot(pooled, fcw_ref[...],
                               preferred_element_type=jnp.float32) + fcb_ref[...]


def kernel(x, edge_index, batch, W1, att_src1, att_dst1, b1,
           W2, att_src2, att_dst2, b2, fcW, fcb):
    f32 = jnp.float32
    W1p = W1.reshape(84, 32, 2).transpose(0, 2, 1).reshape(84, 64)
    W1pp = jnp.pad(W1p, ((0, 44), (0, 0)))
    A1s = att_src1.T.reshape(1, 64)
    A1d = att_dst1.T.reshape(1, 64)
    b1p = b1.reshape(32, 2).T.reshape(1, 64)
    W2p = W2.reshape(32, 2, 64).transpose(1, 0, 2).reshape(64, 64)
    A2s = att_src2.reshape(1, 64)
    A2d = att_dst2.reshape(1, 64)
    b2r = b2.reshape(1, 64)

    x_pad = jnp.pad(x, ((0, NPAD - N), (0, 128 - x.shape[1])))

    loop = jnp.arange(N, dtype=jnp.int32)
    src = jnp.concatenate([edge_index[0].astype(jnp.int32), loop])
    dst = jnp.concatenate([edge_index[1].astype(jnp.int32), loop])
    batch_pad = jnp.concatenate(
        [batch.astype(jnp.int32), jnp.full((NPAD - N,), G, jnp.int32)]
    ).reshape(NBLK, 1, BLKROWS)

    t1 = pl.pallas_call(
        _prep1_body,
        grid=(NBLK,),
        in_specs=[
            pl.BlockSpec((BLKROWS, 128), lambda i: (i, 0)),
            pl.BlockSpec((128, 64), lambda i: (0, 0)),
            pl.BlockSpec((1, 64), lambda i: (0, 0)),
            pl.BlockSpec((1, 64), lambda i: (0, 0)),
        ],
        out_specs=pl.BlockSpec((BLKROWS, W), lambda i: (i, 0)),
        out_shape=jax.ShapeDtypeStruct((NPAD, W), f32),
    )(x_pad, W1pp, A1s, A1d)

    acc1 = _make_edge_kernel(_edge_body1)(t1, src, dst)

    t2 = pl.pallas_call(
        _comb1_body,
        grid=(NBLK,),
        in_specs=[
            pl.BlockSpec((2, BLKROWS, AW), lambda i: (0, i, 0)),
            pl.BlockSpec((64, 64), lambda i: (0, 0)),
            pl.BlockSpec((1, 64), lambda i: (0, 0)),
            pl.BlockSpec((1, 64), lambda i: (0, 0)),
            pl.BlockSpec((1, 64), lambda i: (0, 0)),
        ],
        out_specs=pl.BlockSpec((BLKROWS, W), lambda i: (i, 0)),
        out_shape=jax.ShapeDtypeStruct((NPAD, W), f32),
    )(acc1, W2p, b1p, A2s, A2d)

    acc2 = _make_edge_kernel(_edge_body2)(t2, src, dst)

    out = pl.pallas_call(
        _pool_body,
        grid=(NBLK,),
        in_specs=[
            pl.BlockSpec((2, BLKROWS, AW), lambda i: (0, i, 0)),
            pl.BlockSpec((1, 1, BLKROWS), lambda i: (i, 0, 0)),
            pl.BlockSpec((1, 64), lambda i: (0, 0)),
            pl.BlockSpec((64, 1), lambda i: (0, 0)),
            pl.BlockSpec((1, 1), lambda i: (0, 0)),
        ],
        out_specs=pl.BlockSpec((G, 1), lambda i: (0, 0)),
        out_shape=jax.ShapeDtypeStruct((G, 1), f32),
        scratch_shapes=[pltpu.VMEM((G, 65), f32)],
    )(acc2, batch_pad, b2r, fcW, fcb.reshape(1, 1))

    z = jnp.zeros((1,), f32)
    return (out, z, z, z, z, z, z)
